# Initial kernel scaffold; baseline (speedup 1.0000x reference)
import jax
import jax.numpy as jnp
from jax.experimental import pallas as pl


def kernel(z_batch, slab):
    raise NotImplementedError("write your pallas kernel here")


# speedup vs baseline: 12.6843x; 12.6843x over previous
import jax
import jax.numpy as jnp
from jax import lax
from jax.experimental import pallas as pl
from jax.experimental.pallas import tpu as pltpu

_D = 64
_N = 8
_OUT = 10
_R = 128

_CONV_ROWS = 352
_WDEC_R = 192
_BQ_R, _BK_R, _BV_R, _BCOMB_R = 320, 328, 336, 344
_W1_R = 2 * _CONV_ROWS
_W2_R = _W1_R + 3 * _D
_B1_R = _W2_R + 128
_B2_R = _B1_R + _N

_W4A, _W4B = 0, 64
_WBA, _WBB = 128, 192
_P_W1 = 256
_P_W2 = 448
_P_BIAS = 576
_WROWS = 584


def _pack_weights(slab):
    def pad256(a):
        return jnp.pad(a, ((0, 0), (0, 256 - a.shape[1])))

    def conv_parts(base):
        wq = slab[base + 0:base + 64, 0:_D]
        wk = slab[base + 64:base + 128, 0:_D]
        wv = slab[base + 128:base + 192, 0:_D]
        wtop = slab[base + _WDEC_R:base + _WDEC_R + _D, 0:_D]
        wbot = slab[base + _WDEC_R + _D:base + _WDEC_R + 2 * _D, 0:_D]
        w4 = jnp.concatenate([wq, wk, wv, wtop], axis=1)
        bias4 = jnp.concatenate(
            [slab[base + r, 0:_D] for r in (_BQ_R, _BK_R, _BV_R, _BCOMB_R)])
        return w4, pad256(wbot), bias4[None, :]

    w4_1, wbot_1, b4_1 = conv_parts(0)
    w4_2, wbot_2, b4_2 = conv_parts(_CONV_ROWS)
    w1 = pad256(slab[_W1_R:_W1_R + 3 * _D, :])
    w2 = pad256(slab[_W2_R:_W2_R + 128, :])
    b1 = pad256(slab[_B1_R:_B1_R + 1, :])
    b2 = pad256(slab[_B2_R:_B2_R + 1, :])
    bias_rows = jnp.concatenate(
        [b4_1, b4_2, b1, b2, jnp.zeros((4, 256), jnp.float32)], axis=0)
    return jnp.concatenate(
        [w4_1, w4_2, wbot_1, wbot_2, w1, w2, bias_rows], axis=0)


def _body(x_ref, w_ref, out_ref):
    f32 = jnp.float32
    x = x_ref[...]

    r = lax.broadcasted_iota(jnp.int32, (_R, _R), 0)
    c = lax.broadcasted_iota(jnp.int32, (_R, _R), 1)
    mask = (r // _N) == (c // _N)

    def graph_conv(xin, w4_row, wbot_row, bias_idx):
        qkvt = (jnp.dot(xin, w_ref[w4_row:w4_row + _D, :],
                        preferred_element_type=f32)
                + w_ref[_P_BIAS + bias_idx:_P_BIAS + bias_idx + 1, :])
        q = qkvt[:, 0:_D]
        k = qkvt[:, _D:2 * _D]
        v = qkvt[:, 2 * _D:3 * _D]
        xt = qkvt[:, 3 * _D:4 * _D]
        s = lax.dot_general(q, k, (((1,), (1,)), ((), ())),
                            preferred_element_type=f32)
        s = jnp.where(mask, s, f32(-1e30))
        m = jnp.max(s, axis=-1, keepdims=True)
        e = jnp.exp(s - m)
        attn = e / jnp.sum(e, axis=-1, keepdims=True)
        ctx = jnp.dot(attn, v, preferred_element_type=f32)
        pre = xt + jnp.dot(ctx, w_ref[wbot_row:wbot_row + _D, 0:_D],
                           preferred_element_type=f32)
        return jnp.maximum(pre, 0.0)

    h1 = graph_conv(x, _W4A, _WBA, 0)
    h2 = graph_conv(h1, _W4B, _WBB, 1)

    acc = jnp.dot(x, w_ref[_P_W1:_P_W1 + _D, 0:128],
                  preferred_element_type=f32)
    acc = acc + jnp.dot(h1, w_ref[_P_W1 + _D:_P_W1 + 2 * _D, 0:128],
                        preferred_element_type=f32)
    acc = acc + jnp.dot(h2, w_ref[_P_W1 + 2 * _D:_P_W1 + 3 * _D, 0:128],
                        preferred_element_type=f32)
    acc = acc + w_ref[_P_BIAS + 2:_P_BIAS + 3, 0:128]
    a = jnp.maximum(acc, 0.0)

    logits = (jnp.dot(a, w_ref[_P_W2:_P_W2 + 128, 0:_OUT],
                      preferred_element_type=f32)
              + w_ref[_P_BIAS + 3:_P_BIAS + 4, 0:_OUT])

    m = jnp.max(logits, axis=-1, keepdims=True)
    e = jnp.exp(logits - m)
    thresh = 0.1 * jnp.sum(e, axis=-1, keepdims=True)
    out_ref[...] = jnp.where(e >= thresh, 1.0, 0.0).astype(out_ref.dtype)


@jax.jit
def kernel(z_batch, slab):
    b = z_batch.shape[0]
    graphs_per_block = _R // _N
    b_pad = ((b + graphs_per_block - 1) // graphs_per_block) * graphs_per_block
    z = z_batch
    if b_pad != b:
        z = jnp.pad(z, ((0, b_pad - b), (0, 0), (0, 0)))
    rows = b_pad * _N
    x = z.reshape(rows, _D)
    wpack = _pack_weights(slab)

    flops_per_row = 2 * (64 * 256 + 64 * _R + _R * 64 + 64 * 64) * 2 \
        + 2 * (3 * 64 * 128 + 128 * _OUT)
    out = pl.pallas_call(
        _body,
        grid=(rows // _R,),
        in_specs=[
            pl.BlockSpec((_R, _D), lambda i: (i, 0)),
            pl.BlockSpec((_WROWS, 256), lambda i: (0, 0)),
        ],
        out_specs=pl.BlockSpec((_R, _OUT), lambda i: (i, 0)),
        out_shape=jax.ShapeDtypeStruct((rows, _OUT), jnp.float32),
        compiler_params=pltpu.CompilerParams(
            dimension_semantics=("parallel",)),
        cost_estimate=pl.CostEstimate(
            flops=rows * flops_per_row,
            transcendentals=rows * (_R + _OUT),
            bytes_accessed=_WROWS * 256 * 4 + rows * (_D + _OUT) * 4),
    )(x, wpack)
    return out[:b * _N].reshape(b, _N, _OUT)


# speedup vs baseline: 14.0856x; 1.1105x over previous
import jax
import jax.numpy as jnp
from jax import lax
from jax.experimental import pallas as pl
from jax.experimental.pallas import tpu as pltpu

_D = 64
_N = 8
_OUT = 10
_R = 128
_CHAINS = 4
_TOTAL = _R * _CHAINS

_CONV_ROWS = 352
_WDEC_R = 192
_BQ_R, _BK_R, _BV_R, _BCOMB_R = 320, 328, 336, 344
_W1_R = 2 * _CONV_ROWS
_W2_R = _W1_R + 3 * _D
_B1_R = _W2_R + 128
_B2_R = _B1_R + _N

_W4A, _W4B = 0, 64
_WBA, _WBB = 128, 192
_P_W1 = 256
_P_W2 = 448
_P_BIAS = 576
_WROWS = 584


def _pack_weights(slab):
    def pad256(a):
        return jnp.pad(a, ((0, 0), (0, 256 - a.shape[1])))

    def conv_parts(base):
        wq = slab[base + 0:base + 64, 0:_D]
        wk = slab[base + 64:base + 128, 0:_D]
        wv = slab[base + 128:base + 192, 0:_D]
        wtop = slab[base + _WDEC_R:base + _WDEC_R + _D, 0:_D]
        wbot = slab[base + _WDEC_R + _D:base + _WDEC_R + 2 * _D, 0:_D]
        w4 = jnp.concatenate([wq, wk, wv, wtop], axis=1)
        bias4 = jnp.concatenate(
            [slab[base + r, 0:_D] for r in (_BQ_R, _BK_R, _BV_R, _BCOMB_R)])
        return w4, pad256(wbot), bias4[None, :]

    w4_1, wbot_1, b4_1 = conv_parts(0)
    w4_2, wbot_2, b4_2 = conv_parts(_CONV_ROWS)
    w1 = pad256(slab[_W1_R:_W1_R + 3 * _D, :])
    w2 = pad256(slab[_W2_R:_W2_R + 128, :])
    b1 = pad256(slab[_B1_R:_B1_R + 1, :])
    b2 = pad256(slab[_B2_R:_B2_R + 1, :])
    bias_rows = jnp.concatenate(
        [b4_1, b4_2, b1, b2, jnp.zeros((4, 256), jnp.float32)], axis=0)
    return jnp.concatenate(
        [w4_1, w4_2, wbot_1, wbot_2, w1, w2, bias_rows], axis=0)


def _body(x_ref, w_ref, out_ref):
    f32 = jnp.float32

    r = lax.broadcasted_iota(jnp.int32, (_R, _R), 0)
    c = lax.broadcasted_iota(jnp.int32, (_R, _R), 1)
    mask = (r // _N) == (c // _N)

    def graph_conv(xin, w4_row, wbot_row, bias_idx):
        qkvt = (jnp.dot(xin, w_ref[w4_row:w4_row + _D, :],
                        preferred_element_type=f32)
                + w_ref[_P_BIAS + bias_idx:_P_BIAS + bias_idx + 1, :])
        q = qkvt[:, 0:_D]
        k = qkvt[:, _D:2 * _D]
        v = qkvt[:, 2 * _D:3 * _D]
        xt = qkvt[:, 3 * _D:4 * _D]
        s = lax.dot_general(q, k, (((1,), (1,)), ((), ())),
                            preferred_element_type=f32)
        s = jnp.where(mask, s, f32(-1e30))
        m = jnp.max(s, axis=-1, keepdims=True)
        e = jnp.exp(s - m)
        attn = e / jnp.sum(e, axis=-1, keepdims=True)
        ctx = jnp.dot(attn, v, preferred_element_type=f32)
        pre = xt + jnp.dot(ctx, w_ref[wbot_row:wbot_row + _D, 0:_D],
                           preferred_element_type=f32)
        return jnp.maximum(pre, 0.0)

    for ci in range(_CHAINS):
        x = x_ref[ci * _R:(ci + 1) * _R, :]
        h1 = graph_conv(x, _W4A, _WBA, 0)
        h2 = graph_conv(h1, _W4B, _WBB, 1)

        acc = jnp.dot(x, w_ref[_P_W1:_P_W1 + _D, 0:128],
                      preferred_element_type=f32)
        acc = acc + jnp.dot(h1, w_ref[_P_W1 + _D:_P_W1 + 2 * _D, 0:128],
                            preferred_element_type=f32)
        acc = acc + jnp.dot(h2, w_ref[_P_W1 + 2 * _D:_P_W1 + 3 * _D, 0:128],
                            preferred_element_type=f32)
        acc = acc + w_ref[_P_BIAS + 2:_P_BIAS + 3, 0:128]
        a = jnp.maximum(acc, 0.0)

        logits = (jnp.dot(a, w_ref[_P_W2:_P_W2 + 128, 0:_OUT],
                          preferred_element_type=f32)
                  + w_ref[_P_BIAS + 3:_P_BIAS + 4, 0:_OUT])

        m = jnp.max(logits, axis=-1, keepdims=True)
        e = jnp.exp(logits - m)
        thresh = 0.1 * jnp.sum(e, axis=-1, keepdims=True)
        out_ref[ci * _R:(ci + 1) * _R, :] = jnp.where(
            e >= thresh, 1.0, 0.0).astype(out_ref.dtype)


@jax.jit
def kernel(z_batch, slab):
    b = z_batch.shape[0]
    graphs_per_block = _TOTAL // _N
    b_pad = ((b + graphs_per_block - 1) // graphs_per_block) * graphs_per_block
    z = z_batch
    if b_pad != b:
        z = jnp.pad(z, ((0, b_pad - b), (0, 0), (0, 0)))
    rows = b_pad * _N
    x = z.reshape(rows, _D)
    wpack = _pack_weights(slab)

    flops_per_row = 2 * (64 * 256 + 64 * _R + _R * 64 + 64 * 64) * 2 \
        + 2 * (3 * 64 * 128 + 128 * _OUT)
    out = pl.pallas_call(
        _body,
        grid=(rows // _TOTAL,),
        in_specs=[
            pl.BlockSpec((_TOTAL, _D), lambda i: (i, 0)),
            pl.BlockSpec((_WROWS, 256), lambda i: (0, 0)),
        ],
        out_specs=pl.BlockSpec((_TOTAL, _OUT), lambda i: (i, 0)),
        out_shape=jax.ShapeDtypeStruct((rows, _OUT), jnp.float32),
        compiler_params=pltpu.CompilerParams(
            dimension_semantics=("parallel",)),
        cost_estimate=pl.CostEstimate(
            flops=rows * flops_per_row,
            transcendentals=rows * (_R + _OUT),
            bytes_accessed=_WROWS * 256 * 4 + rows * (_D + _OUT) * 4),
    )(x, wpack)
    return out[:b * _N].reshape(b, _N, _OUT)


# speedup vs baseline: 41.8813x; 2.9733x over previous
import jax
import jax.numpy as jnp
from jax import lax
from jax.experimental import pallas as pl
from jax.experimental.pallas import tpu as pltpu

_D = 64
_N = 8
_OUT = 10
_R = 128
_CHAINS = 4
_TOTAL = _R * _CHAINS

_CONV_ROWS = 352
_WDEC_R = 192
_BQ_R, _BK_R, _BV_R, _BCOMB_R = 320, 328, 336, 344
_W1_R = 2 * _CONV_ROWS
_W2_R = _W1_R + 3 * _D
_B1_R = _W2_R + 128
_B2_R = _B1_R + _N

_W4A, _W4B = 0, 64
_WBA, _WBB = 128, 192
_P_W1 = 256
_P_W2 = 448
_P_BIAS = 576
_WROWS = 584


def _pack_weights(slab):
    def pad256(a):
        return jnp.pad(a, ((0, 0), (0, 256 - a.shape[1])))

    def conv_parts(base):
        wq = slab[base + 0:base + 64, 0:_D]
        wk = slab[base + 64:base + 128, 0:_D]
        wv = slab[base + 128:base + 192, 0:_D]
        wtop = slab[base + _WDEC_R:base + _WDEC_R + _D, 0:_D]
        wbot = slab[base + _WDEC_R + _D:base + _WDEC_R + 2 * _D, 0:_D]
        w4 = jnp.concatenate([wq, wk, wv, wtop], axis=1)
        bias4 = jnp.concatenate(
            [slab[base + r, 0:_D] for r in (_BQ_R, _BK_R, _BV_R, _BCOMB_R)])
        return w4, pad256(wbot), bias4[None, :]

    w4_1, wbot_1, b4_1 = conv_parts(0)
    w4_2, wbot_2, b4_2 = conv_parts(_CONV_ROWS)
    w1 = pad256(slab[_W1_R:_W1_R + 3 * _D, :])
    w2 = pad256(slab[_W2_R:_W2_R + 128, :])
    b1 = pad256(slab[_B1_R:_B1_R + 1, :])
    b2 = pad256(slab[_B2_R:_B2_R + 1, :])
    bias_rows = jnp.concatenate(
        [b4_1, b4_2, b1, b2, jnp.zeros((4, 256), jnp.float32)], axis=0)
    return jnp.concatenate(
        [w4_1, w4_2, wbot_1, wbot_2, w1, w2, bias_rows], axis=0)


def _body(x_ref, w_ref, out_ref):
    f32 = jnp.float32

    r = lax.broadcasted_iota(jnp.int32, (_R, _R), 0)
    c = lax.broadcasted_iota(jnp.int32, (_R, _R), 1)
    mask = (r // _N) == (c // _N)

    def graph_conv(xin, w4_row, bias_idx):
        wbot_row = _WBA if w4_row == _W4A else _WBB
        w4 = w_ref[w4_row:w4_row + _D, :]
        bias = w_ref[_P_BIAS + bias_idx:_P_BIAS + bias_idx + 1, :]
        wbot = w_ref[wbot_row:wbot_row + _D, 0:_D]

        qkvt = [jnp.dot(xc, w4, preferred_element_type=f32) + bias
                for xc in xin]
        s = [lax.dot_general(t[:, 0:_D], t[:, _D:2 * _D],
                             (((1,), (1,)), ((), ())),
                             preferred_element_type=f32) for t in qkvt]
        s = [jnp.where(mask, sc, f32(-1e30)) for sc in s]
        m = [jnp.max(sc, axis=-1, keepdims=True) for sc in s]
        e = [jnp.exp(sc - mc) for sc, mc in zip(s, m)]
        attn = [ec / jnp.sum(ec, axis=-1, keepdims=True) for ec in e]
        ctx = [jnp.dot(ac, t[:, 2 * _D:3 * _D], preferred_element_type=f32)
               for ac, t in zip(attn, qkvt)]
        pre = [t[:, 3 * _D:4 * _D]
               + jnp.dot(cc, wbot, preferred_element_type=f32)
               for cc, t in zip(ctx, qkvt)]
        return [jnp.maximum(p, 0.0) for p in pre]

    xs = [x_ref[ci * _R:(ci + 1) * _R, :] for ci in range(_CHAINS)]
    h1 = graph_conv(xs, _W4A, 0)
    h2 = graph_conv(h1, _W4B, 1)

    w1a = w_ref[_P_W1:_P_W1 + _D, 0:128]
    w1b = w_ref[_P_W1 + _D:_P_W1 + 2 * _D, 0:128]
    w1c = w_ref[_P_W1 + 2 * _D:_P_W1 + 3 * _D, 0:128]
    b1 = w_ref[_P_BIAS + 2:_P_BIAS + 3, 0:128]
    w2 = w_ref[_P_W2:_P_W2 + 128, 0:_OUT]
    b2 = w_ref[_P_BIAS + 3:_P_BIAS + 4, 0:_OUT]

    acc = [jnp.dot(xc, w1a, preferred_element_type=f32) for xc in xs]
    acc = [a + jnp.dot(h, w1b, preferred_element_type=f32)
           for a, h in zip(acc, h1)]
    acc = [a + jnp.dot(h, w1c, preferred_element_type=f32)
           for a, h in zip(acc, h2)]
    a = [jnp.maximum(ac + b1, 0.0) for ac in acc]

    logits = [jnp.dot(ac, w2, preferred_element_type=f32) + b2 for ac in a]
    m = [jnp.max(lg, axis=-1, keepdims=True) for lg in logits]
    e = [jnp.exp(lg - mc) for lg, mc in zip(logits, m)]
    thresh = [0.1 * jnp.sum(ec, axis=-1, keepdims=True) for ec in e]
    for ci in range(_CHAINS):
        out_ref[ci * _R:(ci + 1) * _R, :] = jnp.where(
            e[ci] >= thresh[ci], 1.0, 0.0).astype(out_ref.dtype)


@jax.jit
def kernel(z_batch, slab):
    b = z_batch.shape[0]
    graphs_per_block = _TOTAL // _N
    b_pad = ((b + graphs_per_block - 1) // graphs_per_block) * graphs_per_block
    z = z_batch
    if b_pad != b:
        z = jnp.pad(z, ((0, b_pad - b), (0, 0), (0, 0)))
    rows = b_pad * _N
    x = z.reshape(rows, _D)
    wpack = _pack_weights(slab)

    flops_per_row = 2 * (64 * 256 + 64 * _R + _R * 64 + 64 * 64) * 2 \
        + 2 * (3 * 64 * 128 + 128 * _OUT)
    out = pl.pallas_call(
        _body,
        grid=(rows // _TOTAL,),
        in_specs=[
            pl.BlockSpec((_TOTAL, _D), lambda i: (i, 0)),
            pl.BlockSpec((_WROWS, 256), lambda i: (0, 0)),
        ],
        out_specs=pl.BlockSpec((_TOTAL, _OUT), lambda i: (i, 0)),
        out_shape=jax.ShapeDtypeStruct((rows, _OUT), jnp.float32),
        compiler_params=pltpu.CompilerParams(
            dimension_semantics=("parallel",)),
        cost_estimate=pl.CostEstimate(
            flops=rows * flops_per_row,
            transcendentals=rows * (_R + _OUT),
            bytes_accessed=_WROWS * 256 * 4 + rows * (_D + _OUT) * 4),
    )(x, wpack)
    return out[:b * _N].reshape(b, _N, _OUT)


# speedup vs baseline: 65.0882x; 1.5541x over previous
import jax
import jax.numpy as jnp
from jax import lax
from jax.experimental import pallas as pl
from jax.experimental.pallas import tpu as pltpu

_D = 64
_N = 8
_OUT = 10
_R = 128
_CHAINS = 8
_TOTAL = _R * _CHAINS

_CONV_ROWS = 352
_WDEC_R = 192
_BQ_R, _BK_R, _BV_R, _BCOMB_R = 320, 328, 336, 344
_W1_R = 2 * _CONV_ROWS
_W2_R = _W1_R + 3 * _D
_B1_R = _W2_R + 128
_B2_R = _B1_R + _N

_W4A, _W4B = 0, 64
_WBA, _WBB = 128, 192
_P_W1 = 256
_P_W2 = 448
_P_BIAS = 576
_WROWS = 584


def _pack_weights(slab):
    def pad256(a):
        return jnp.pad(a, ((0, 0), (0, 256 - a.shape[1])))

    def conv_parts(base):
        wq = slab[base + 0:base + 64, 0:_D]
        wk = slab[base + 64:base + 128, 0:_D]
        wv = slab[base + 128:base + 192, 0:_D]
        wtop = slab[base + _WDEC_R:base + _WDEC_R + _D, 0:_D]
        wbot = slab[base + _WDEC_R + _D:base + _WDEC_R + 2 * _D, 0:_D]
        w4 = jnp.concatenate([wq, wk, wv, wtop], axis=1)
        bias4 = jnp.concatenate(
            [slab[base + r, 0:_D] for r in (_BQ_R, _BK_R, _BV_R, _BCOMB_R)])
        return w4, pad256(wbot), bias4[None, :]

    w4_1, wbot_1, b4_1 = conv_parts(0)
    w4_2, wbot_2, b4_2 = conv_parts(_CONV_ROWS)
    w1 = pad256(slab[_W1_R:_W1_R + 3 * _D, :])
    w2 = pad256(slab[_W2_R:_W2_R + 128, :])
    b1 = pad256(slab[_B1_R:_B1_R + 1, :])
    b2 = pad256(slab[_B2_R:_B2_R + 1, :])
    bias_rows = jnp.concatenate(
        [b4_1, b4_2, b1, b2, jnp.zeros((4, 256), jnp.float32)], axis=0)
    return jnp.concatenate(
        [w4_1, w4_2, wbot_1, wbot_2, w1, w2, bias_rows], axis=0)


def _body(x_ref, w_ref, out_ref):
    f32 = jnp.float32

    r = lax.broadcasted_iota(jnp.int32, (_R, _R), 0)
    c = lax.broadcasted_iota(jnp.int32, (_R, _R), 1)
    mask = (r // _N) == (c // _N)

    def graph_conv(xin, w4_row, bias_idx):
        wbot_row = _WBA if w4_row == _W4A else _WBB
        w4 = w_ref[w4_row:w4_row + _D, :]
        bias = w_ref[_P_BIAS + bias_idx:_P_BIAS + bias_idx + 1, :]
        wbot = w_ref[wbot_row:wbot_row + _D, 0:_D]

        qkvt = [jnp.dot(xc, w4, preferred_element_type=f32) + bias
                for xc in xin]
        s = [lax.dot_general(t[:, 0:_D], t[:, _D:2 * _D],
                             (((1,), (1,)), ((), ())),
                             preferred_element_type=f32) for t in qkvt]
        s = [jnp.where(mask, sc, f32(-1e30)) for sc in s]
        m = [jnp.max(sc, axis=-1, keepdims=True) for sc in s]
        e = [jnp.exp(sc - mc) for sc, mc in zip(s, m)]
        attn = [ec / jnp.sum(ec, axis=-1, keepdims=True) for ec in e]
        ctx = [jnp.dot(ac, t[:, 2 * _D:3 * _D], preferred_element_type=f32)
               for ac, t in zip(attn, qkvt)]
        pre = [t[:, 3 * _D:4 * _D]
               + jnp.dot(cc, wbot, preferred_element_type=f32)
               for cc, t in zip(ctx, qkvt)]
        return [jnp.maximum(p, 0.0) for p in pre]

    xs = [x_ref[ci * _R:(ci + 1) * _R, :] for ci in range(_CHAINS)]
    h1 = graph_conv(xs, _W4A, 0)
    h2 = graph_conv(h1, _W4B, 1)

    w1a = w_ref[_P_W1:_P_W1 + _D, 0:128]
    w1b = w_ref[_P_W1 + _D:_P_W1 + 2 * _D, 0:128]
    w1c = w_ref[_P_W1 + 2 * _D:_P_W1 + 3 * _D, 0:128]
    b1 = w_ref[_P_BIAS + 2:_P_BIAS + 3, 0:128]
    w2 = w_ref[_P_W2:_P_W2 + 128, 0:_OUT]
    b2 = w_ref[_P_BIAS + 3:_P_BIAS + 4, 0:_OUT]

    acc = [jnp.dot(xc, w1a, preferred_element_type=f32) for xc in xs]
    acc = [a + jnp.dot(h, w1b, preferred_element_type=f32)
           for a, h in zip(acc, h1)]
    acc = [a + jnp.dot(h, w1c, preferred_element_type=f32)
           for a, h in zip(acc, h2)]
    a = [jnp.maximum(ac + b1, 0.0) for ac in acc]

    logits = [jnp.dot(ac, w2, preferred_element_type=f32) + b2 for ac in a]
    m = [jnp.max(lg, axis=-1, keepdims=True) for lg in logits]
    e = [jnp.exp(lg - mc) for lg, mc in zip(logits, m)]
    thresh = [0.1 * jnp.sum(ec, axis=-1, keepdims=True) for ec in e]
    for ci in range(_CHAINS):
        out_ref[ci * _R:(ci + 1) * _R, :] = jnp.where(
            e[ci] >= thresh[ci], 1.0, 0.0).astype(out_ref.dtype)


@jax.jit
def kernel(z_batch, slab):
    b = z_batch.shape[0]
    graphs_per_block = _TOTAL // _N
    b_pad = ((b + graphs_per_block - 1) // graphs_per_block) * graphs_per_block
    z = z_batch
    if b_pad != b:
        z = jnp.pad(z, ((0, b_pad - b), (0, 0), (0, 0)))
    rows = b_pad * _N
    x = z.reshape(rows, _D)
    wpack = _pack_weights(slab)

    flops_per_row = 2 * (64 * 256 + 64 * _R + _R * 64 + 64 * 64) * 2 \
        + 2 * (3 * 64 * 128 + 128 * _OUT)
    out = pl.pallas_call(
        _body,
        grid=(rows // _TOTAL,),
        in_specs=[
            pl.BlockSpec((_TOTAL, _D), lambda i: (i, 0)),
            pl.BlockSpec((_WROWS, 256), lambda i: (0, 0)),
        ],
        out_specs=pl.BlockSpec((_TOTAL, _OUT), lambda i: (i, 0)),
        out_shape=jax.ShapeDtypeStruct((rows, _OUT), jnp.float32),
        compiler_params=pltpu.CompilerParams(
            dimension_semantics=("parallel",)),
        cost_estimate=pl.CostEstimate(
            flops=rows * flops_per_row,
            transcendentals=rows * (_R + _OUT),
            bytes_accessed=_WROWS * 256 * 4 + rows * (_D + _OUT) * 4),
    )(x, wpack)
    return out[:b * _N].reshape(b, _N, _OUT)


# speedup vs baseline: 81.1911x; 1.2474x over previous
import jax
import jax.numpy as jnp
from jax import lax
from jax.experimental import pallas as pl
from jax.experimental.pallas import tpu as pltpu

_D = 64
_N = 8
_OUT = 10
_R = 128
_CHAINS = 16
_TOTAL = _R * _CHAINS

_CONV_ROWS = 352
_WDEC_R = 192
_BQ_R, _BK_R, _BV_R, _BCOMB_R = 320, 328, 336, 344
_W1_R = 2 * _CONV_ROWS
_W2_R = _W1_R + 3 * _D
_B1_R = _W2_R + 128
_B2_R = _B1_R + _N

_W4A, _W4B = 0, 64
_WBA, _WBB = 128, 192
_P_W1 = 256
_P_W2 = 448
_P_BIAS = 576
_WROWS = 584


def _pack_weights(slab):
    def pad256(a):
        return jnp.pad(a, ((0, 0), (0, 256 - a.shape[1])))

    def conv_parts(base):
        wq = slab[base + 0:base + 64, 0:_D]
        wk = slab[base + 64:base + 128, 0:_D]
        wv = slab[base + 128:base + 192, 0:_D]
        wtop = slab[base + _WDEC_R:base + _WDEC_R + _D, 0:_D]
        wbot = slab[base + _WDEC_R + _D:base + _WDEC_R + 2 * _D, 0:_D]
        w4 = jnp.concatenate([wq, wk, wv, wtop], axis=1)
        bias4 = jnp.concatenate(
            [slab[base + r, 0:_D] for r in (_BQ_R, _BK_R, _BV_R, _BCOMB_R)])
        return w4, pad256(wbot), bias4[None, :]

    w4_1, wbot_1, b4_1 = conv_parts(0)
    w4_2, wbot_2, b4_2 = conv_parts(_CONV_ROWS)
    w1 = pad256(slab[_W1_R:_W1_R + 3 * _D, :])
    w2 = pad256(slab[_W2_R:_W2_R + 128, :])
    b1 = pad256(slab[_B1_R:_B1_R + 1, :])
    b2 = pad256(slab[_B2_R:_B2_R + 1, :])
    bias_rows = jnp.concatenate(
        [b4_1, b4_2, b1, b2, jnp.zeros((4, 256), jnp.float32)], axis=0)
    return jnp.concatenate(
        [w4_1, w4_2, wbot_1, wbot_2, w1, w2, bias_rows], axis=0)


def _body(x_ref, w_ref, out_ref):
    f32 = jnp.float32

    r = lax.broadcasted_iota(jnp.int32, (_R, _R), 0)
    c = lax.broadcasted_iota(jnp.int32, (_R, _R), 1)
    mask = (r // _N) == (c // _N)

    def graph_conv(xin, w4_row, bias_idx):
        wbot_row = _WBA if w4_row == _W4A else _WBB
        w4 = w_ref[w4_row:w4_row + _D, :]
        bias = w_ref[_P_BIAS + bias_idx:_P_BIAS + bias_idx + 1, :]
        wbot = w_ref[wbot_row:wbot_row + _D, 0:_D]

        qkvt = [jnp.dot(xc, w4, preferred_element_type=f32) + bias
                for xc in xin]
        s = [lax.dot_general(t[:, 0:_D], t[:, _D:2 * _D],
                             (((1,), (1,)), ((), ())),
                             preferred_element_type=f32) for t in qkvt]
        s = [jnp.where(mask, sc, f32(-1e30)) for sc in s]
        m = [jnp.max(sc, axis=-1, keepdims=True) for sc in s]
        e = [jnp.exp(sc - mc) for sc, mc in zip(s, m)]
        attn = [ec / jnp.sum(ec, axis=-1, keepdims=True) for ec in e]
        ctx = [jnp.dot(ac, t[:, 2 * _D:3 * _D], preferred_element_type=f32)
               for ac, t in zip(attn, qkvt)]
        pre = [t[:, 3 * _D:4 * _D]
               + jnp.dot(cc, wbot, preferred_element_type=f32)
               for cc, t in zip(ctx, qkvt)]
        return [jnp.maximum(p, 0.0) for p in pre]

    xs = [x_ref[ci * _R:(ci + 1) * _R, :] for ci in range(_CHAINS)]
    h1 = graph_conv(xs, _W4A, 0)
    h2 = graph_conv(h1, _W4B, 1)

    w1a = w_ref[_P_W1:_P_W1 + _D, 0:128]
    w1b = w_ref[_P_W1 + _D:_P_W1 + 2 * _D, 0:128]
    w1c = w_ref[_P_W1 + 2 * _D:_P_W1 + 3 * _D, 0:128]
    b1 = w_ref[_P_BIAS + 2:_P_BIAS + 3, 0:128]
    w2 = w_ref[_P_W2:_P_W2 + 128, 0:_OUT]
    b2 = w_ref[_P_BIAS + 3:_P_BIAS + 4, 0:_OUT]

    acc = [jnp.dot(xc, w1a, preferred_element_type=f32) for xc in xs]
    acc = [a + jnp.dot(h, w1b, preferred_element_type=f32)
           for a, h in zip(acc, h1)]
    acc = [a + jnp.dot(h, w1c, preferred_element_type=f32)
           for a, h in zip(acc, h2)]
    a = [jnp.maximum(ac + b1, 0.0) for ac in acc]

    logits = [jnp.dot(ac, w2, preferred_element_type=f32) + b2 for ac in a]
    m = [jnp.max(lg, axis=-1, keepdims=True) for lg in logits]
    e = [jnp.exp(lg - mc) for lg, mc in zip(logits, m)]
    thresh = [0.1 * jnp.sum(ec, axis=-1, keepdims=True) for ec in e]
    for ci in range(_CHAINS):
        out_ref[ci * _R:(ci + 1) * _R, :] = jnp.where(
            e[ci] >= thresh[ci], 1.0, 0.0).astype(out_ref.dtype)


@jax.jit
def kernel(z_batch, slab):
    b = z_batch.shape[0]
    graphs_per_block = _TOTAL // _N
    b_pad = ((b + graphs_per_block - 1) // graphs_per_block) * graphs_per_block
    z = z_batch
    if b_pad != b:
        z = jnp.pad(z, ((0, b_pad - b), (0, 0), (0, 0)))
    rows = b_pad * _N
    x = z.reshape(rows, _D)
    wpack = _pack_weights(slab)

    flops_per_row = 2 * (64 * 256 + 64 * _R + _R * 64 + 64 * 64) * 2 \
        + 2 * (3 * 64 * 128 + 128 * _OUT)
    out = pl.pallas_call(
        _body,
        grid=(rows // _TOTAL,),
        in_specs=[
            pl.BlockSpec((_TOTAL, _D), lambda i: (i, 0)),
            pl.BlockSpec((_WROWS, 256), lambda i: (0, 0)),
        ],
        out_specs=pl.BlockSpec((_TOTAL, _OUT), lambda i: (i, 0)),
        out_shape=jax.ShapeDtypeStruct((rows, _OUT), jnp.float32),
        compiler_params=pltpu.CompilerParams(
            dimension_semantics=("parallel",)),
        cost_estimate=pl.CostEstimate(
            flops=rows * flops_per_row,
            transcendentals=rows * (_R + _OUT),
            bytes_accessed=_WROWS * 256 * 4 + rows * (_D + _OUT) * 4),
    )(x, wpack)
    return out[:b * _N].reshape(b, _N, _OUT)


# speedup vs baseline: 89.2596x; 1.0994x over previous
import jax
import jax.numpy as jnp
from jax import lax
from jax.experimental import pallas as pl
from jax.experimental.pallas import tpu as pltpu

_D = 64
_N = 8
_OUT = 10
_R = 128
_CHAINS = 32
_TOTAL = _R * _CHAINS

_CONV_ROWS = 352
_WDEC_R = 192
_BQ_R, _BK_R, _BV_R, _BCOMB_R = 320, 328, 336, 344
_W1_R = 2 * _CONV_ROWS
_W2_R = _W1_R + 3 * _D
_B1_R = _W2_R + 128
_B2_R = _B1_R + _N

_W4A, _W4B = 0, 64
_WBA, _WBB = 128, 192
_P_W1 = 256
_P_W2 = 448
_P_BIAS = 576
_WROWS = 584


def _pack_weights(slab):
    def pad256(a):
        return jnp.pad(a, ((0, 0), (0, 256 - a.shape[1])))

    def conv_parts(base):
        wq = slab[base + 0:base + 64, 0:_D]
        wk = slab[base + 64:base + 128, 0:_D]
        wv = slab[base + 128:base + 192, 0:_D]
        wtop = slab[base + _WDEC_R:base + _WDEC_R + _D, 0:_D]
        wbot = slab[base + _WDEC_R + _D:base + _WDEC_R + 2 * _D, 0:_D]
        w4 = jnp.concatenate([wq, wk, wv, wtop], axis=1)
        bias4 = jnp.concatenate(
            [slab[base + r, 0:_D] for r in (_BQ_R, _BK_R, _BV_R, _BCOMB_R)])
        return w4, pad256(wbot), bias4[None, :]

    w4_1, wbot_1, b4_1 = conv_parts(0)
    w4_2, wbot_2, b4_2 = conv_parts(_CONV_ROWS)
    w1 = pad256(slab[_W1_R:_W1_R + 3 * _D, :])
    w2 = pad256(slab[_W2_R:_W2_R + 128, :])
    b1 = pad256(slab[_B1_R:_B1_R + 1, :])
    b2 = pad256(slab[_B2_R:_B2_R + 1, :])
    bias_rows = jnp.concatenate(
        [b4_1, b4_2, b1, b2, jnp.zeros((4, 256), jnp.float32)], axis=0)
    return jnp.concatenate(
        [w4_1, w4_2, wbot_1, wbot_2, w1, w2, bias_rows], axis=0)


def _body(x_ref, w_ref, out_ref):
    f32 = jnp.float32

    r = lax.broadcasted_iota(jnp.int32, (_R, _R), 0)
    c = lax.broadcasted_iota(jnp.int32, (_R, _R), 1)
    mask = (r // _N) == (c // _N)

    def graph_conv(xin, w4_row, bias_idx):
        wbot_row = _WBA if w4_row == _W4A else _WBB
        w4 = w_ref[w4_row:w4_row + _D, :]
        bias = w_ref[_P_BIAS + bias_idx:_P_BIAS + bias_idx + 1, :]
        wbot = w_ref[wbot_row:wbot_row + _D, 0:_D]

        qkvt = [jnp.dot(xc, w4, preferred_element_type=f32) + bias
                for xc in xin]
        s = [lax.dot_general(t[:, 0:_D], t[:, _D:2 * _D],
                             (((1,), (1,)), ((), ())),
                             preferred_element_type=f32) for t in qkvt]
        s = [jnp.where(mask, sc, f32(-1e30)) for sc in s]
        m = [jnp.max(sc, axis=-1, keepdims=True) for sc in s]
        e = [jnp.exp(sc - mc) for sc, mc in zip(s, m)]
        attn = [ec / jnp.sum(ec, axis=-1, keepdims=True) for ec in e]
        ctx = [jnp.dot(ac, t[:, 2 * _D:3 * _D], preferred_element_type=f32)
               for ac, t in zip(attn, qkvt)]
        pre = [t[:, 3 * _D:4 * _D]
               + jnp.dot(cc, wbot, preferred_element_type=f32)
               for cc, t in zip(ctx, qkvt)]
        return [jnp.maximum(p, 0.0) for p in pre]

    xs = [x_ref[ci * _R:(ci + 1) * _R, :] for ci in range(_CHAINS)]
    h1 = graph_conv(xs, _W4A, 0)
    h2 = graph_conv(h1, _W4B, 1)

    w1a = w_ref[_P_W1:_P_W1 + _D, 0:128]
    w1b = w_ref[_P_W1 + _D:_P_W1 + 2 * _D, 0:128]
    w1c = w_ref[_P_W1 + 2 * _D:_P_W1 + 3 * _D, 0:128]
    b1 = w_ref[_P_BIAS + 2:_P_BIAS + 3, 0:128]
    w2 = w_ref[_P_W2:_P_W2 + 128, 0:_OUT]
    b2 = w_ref[_P_BIAS + 3:_P_BIAS + 4, 0:_OUT]

    acc = [jnp.dot(xc, w1a, preferred_element_type=f32) for xc in xs]
    acc = [a + jnp.dot(h, w1b, preferred_element_type=f32)
           for a, h in zip(acc, h1)]
    acc = [a + jnp.dot(h, w1c, preferred_element_type=f32)
           for a, h in zip(acc, h2)]
    a = [jnp.maximum(ac + b1, 0.0) for ac in acc]

    logits = [jnp.dot(ac, w2, preferred_element_type=f32) + b2 for ac in a]
    m = [jnp.max(lg, axis=-1, keepdims=True) for lg in logits]
    e = [jnp.exp(lg - mc) for lg, mc in zip(logits, m)]
    thresh = [0.1 * jnp.sum(ec, axis=-1, keepdims=True) for ec in e]
    for ci in range(_CHAINS):
        out_ref[ci * _R:(ci + 1) * _R, :] = jnp.where(
            e[ci] >= thresh[ci], 1.0, 0.0).astype(out_ref.dtype)


@jax.jit
def kernel(z_batch, slab):
    b = z_batch.shape[0]
    graphs_per_block = _TOTAL // _N
    b_pad = ((b + graphs_per_block - 1) // graphs_per_block) * graphs_per_block
    z = z_batch
    if b_pad != b:
        z = jnp.pad(z, ((0, b_pad - b), (0, 0), (0, 0)))
    rows = b_pad * _N
    x = z.reshape(rows, _D)
    wpack = _pack_weights(slab)

    flops_per_row = 2 * (64 * 256 + 64 * _R + _R * 64 + 64 * 64) * 2 \
        + 2 * (3 * 64 * 128 + 128 * _OUT)
    out = pl.pallas_call(
        _body,
        grid=(rows // _TOTAL,),
        in_specs=[
            pl.BlockSpec((_TOTAL, _D), lambda i: (i, 0)),
            pl.BlockSpec((_WROWS, 256), lambda i: (0, 0)),
        ],
        out_specs=pl.BlockSpec((_TOTAL, _OUT), lambda i: (i, 0)),
        out_shape=jax.ShapeDtypeStruct((rows, _OUT), jnp.float32),
        compiler_params=pltpu.CompilerParams(
            dimension_semantics=("parallel",)),
        cost_estimate=pl.CostEstimate(
            flops=rows * flops_per_row,
            transcendentals=rows * (_R + _OUT),
            bytes_accessed=_WROWS * 256 * 4 + rows * (_D + _OUT) * 4),
    )(x, wpack)
    return out[:b * _N].reshape(b, _N, _OUT)
